# baseline (speedup 1.0000x reference)
import functools

import jax
import jax.numpy as jnp
from jax import lax
from jax.experimental import pallas as pl
from jax.experimental.pallas import tpu as pltpu
from jax.experimental.pallas import tpu_sc as plsc

N = 10000
E = 320000
IN_C = 128
H = 256
HH = 128
G = 64
NC = 2
NS = 16
K = 128
EPT = E // NS
IDXG = 32
NGRP = 5
CT = IDXG * NGRP
EPTP = CT * K
GROWS = 2 * N + 8
DUMMY = N
ZR = 632
ACCR = ZR * NS
DPT = 624

_MESH = plsc.VectorSubcoreMesh(core_axis_name="c", subcore_axis_name="s")


def _dump(acc, out_hbm, c, s):
    @pl.when(s < NS - 1)
    def _():
        pltpu.sync_copy(acc.at[pl.ds(s * DPT, DPT)],
                        out_hbm.at[pl.ds(c * N + s * DPT, DPT)])

    @pl.when(s == NS - 1)
    def _():
        pltpu.sync_copy(acc.at[pl.ds((NS - 1) * DPT, N - (NS - 1) * DPT)],
                        out_hbm.at[pl.ds(c * N + (NS - 1) * DPT,
                                         N - (NS - 1) * DPT)])

@functools.partial(
    pl.kernel,
    out_type=jax.ShapeDtypeStruct((2 * N, HH), jnp.float32),
    mesh=_MESH,
    scratch_types=[
        pltpu.VMEM((IDXG, K), jnp.int32),
        pltpu.VMEM((IDXG, K), jnp.int32),
        pltpu.VMEM((K, HH), jnp.float32),
        pltpu.VMEM((K, HH), jnp.float32),
        pltpu.VMEM_SHARED((ACCR, HH), jnp.float32),
        pltpu.SemaphoreType.DMA,
        pltpu.SemaphoreType.DMA,
        pltpu.SemaphoreType.DMA,
        pltpu.SemaphoreType.DMA,
    ],
)
def _sc_scatter(g_hbm, srcp_hbm, dstp_hbm, z_hbm, out_hbm,
                src_v, dst_v, rows_a, rows_b, acc,
                gsem_a, gsem_b, ssem_a, ssem_b):
    c = lax.axis_index("c")
    s = lax.axis_index("s")
    tid = c * NS + s
    pltpu.sync_copy(z_hbm, acc.at[pl.ds(s * ZR, ZR)])
    plsc.subcore_barrier()

    def group(gi, carry):
        pltpu.sync_copy(srcp_hbm.at[tid, pl.ds(gi * IDXG, IDXG)], src_v)
        pltpu.sync_copy(dstp_hbm.at[s, pl.ds(gi * IDXG, IDXG)], dst_v)
        pltpu.async_copy(g_hbm.at[src_v.at[0]], rows_a, gsem_a)

        def step(j, cur, gsem_cur, ssem_cur, nxt, gsem_nxt, ssem_nxt):
            pltpu.make_async_copy(g_hbm.at[src_v.at[j]], cur,
                                  gsem_cur).wait()

            @pl.when(j + 1 < IDXG)
            def _():
                @pl.when(j >= 1)
                def _():
                    pltpu.make_async_copy(nxt, acc.at[dst_v.at[j]],
                                          ssem_nxt).wait()

                pltpu.async_copy(g_hbm.at[src_v.at[j + 1]], nxt, gsem_nxt)

            pltpu.async_copy(cur, acc.at[dst_v.at[j]], ssem_cur, add=True)

        def pair(k, carry2):
            step(2 * k, rows_a, gsem_a, ssem_a, rows_b, gsem_b, ssem_b)
            step(2 * k + 1, rows_b, gsem_b, ssem_b, rows_a, gsem_a, ssem_a)
            return carry2

        carry = lax.fori_loop(0, IDXG // 2, pair, carry)
        pltpu.make_async_copy(rows_a, acc.at[dst_v.at[0]], ssem_a).wait()
        pltpu.make_async_copy(rows_b, acc.at[dst_v.at[0]], ssem_b).wait()
        return carry

    lax.fori_loop(0, NGRP, group, 0)
    plsc.subcore_barrier()
    _dump(acc, out_hbm, c, s)


_TC_PARAMS = pltpu.CompilerParams(vmem_limit_bytes=60 * 1024 * 1024)

def _dinv_from(dg_ref):
    deg = (dg_ref[0:N, 0:1] + dg_ref[N:2 * N, 0:1]) * 0.5 + 1.0
    return lax.rsqrt(deg)


def _split_cols(ref):
    return jnp.concatenate([ref[0:N, :], ref[N:2 * N, :]], axis=1)


def _store_halves(out_ref, gn):
    out_ref[0:N, :] = gn[:, 0:HH]
    out_ref[N:2 * N, :] = gn[:, HH:H]
    out_ref[2 * N:GROWS, :] = jnp.zeros((GROWS - 2 * N, HH), jnp.float32)


def _split_bf16(x):
    hi = x.astype(jnp.bfloat16)
    lo = (x - hi.astype(jnp.float32)).astype(jnp.bfloat16)
    return hi, lo


def _mm3(x, w):
    xh, xl = _split_bf16(x)
    wh, wl = _split_bf16(w)
    a = jnp.concatenate([xh, xl, xh], axis=1)
    b = jnp.concatenate([wh, wh, wl], axis=0)
    return jnp.dot(a, b, preferred_element_type=jnp.float32)


def _pdot(p_bf, x):
    xh, xl = _split_bf16(x)
    return (jnp.dot(p_bf, xh, preferred_element_type=jnp.float32)
            + jnp.dot(p_bf, xl, preferred_element_type=jnp.float32))


_BCAST_DIMS = (((0,), (0,)), ((), ()))


def _pbcast(p_bf, x):
    xh, xl = _split_bf16(x)
    return (lax.dot_general(p_bf, xh, _BCAST_DIMS,
                            preferred_element_type=jnp.float32)
            + lax.dot_general(p_bf, xl, _BCAST_DIMS,
                              preferred_element_type=jnp.float32))


def _tc_embed_body(x_ref, we_ref, be_ref, w0_ref, dg_ref, out_ref):
    wc = _mm3(we_ref[...], w0_ref[...])
    bc = _mm3(be_ref[...], w0_ref[...])
    g = _mm3(x_ref[...], wc) + bc
    _store_halves(out_ref, g * _dinv_from(dg_ref))


_tc_embed = pl.pallas_call(
    _tc_embed_body,
    out_shape=jax.ShapeDtypeStruct((GROWS, HH), jnp.float32),
    compiler_params=_TC_PARAMS,
)


def _norm_pool_half(agg_ref, gt_ref, dg_ref, batch_ref, b_ref, gw_ref,
                    gb_ref, gs_ref):
    dinv = _dinv_from(dg_ref)
    t = dinv * (agg_ref[...] + gt_ref[...]) + b_ref[...]
    p = (lax.broadcasted_iota(jnp.int32, (G, N), 0)
         == batch_ref[...]).astype(jnp.bfloat16)
    cnt = jnp.maximum(jnp.sum(p.astype(jnp.float32), axis=1,
                              keepdims=True), 1.0)
    mean = _pdot(p, t) / cnt
    meanfull = _pbcast(p, mean)
    o = t - meanfull * gs_ref[...]
    var = _pdot(p, o * o) / cnt
    inv_std = lax.rsqrt(var + 1e-5)
    isf = _pbcast(p, inv_std)
    h = gw_ref[...] * o * isf + gb_ref[...]
    h = jnp.where(h >= 0, h, 0.01 * h)
    feats = _pdot(p, h) / cnt
    return h, feats


def _tc_norm_mid_body(agg_ref, gt_ref, dg_ref, batch_ref, b_ref, gw_ref,
                      gb_ref, gs_ref, h_ref, feats_ref):
    h, feats = _norm_pool_half(agg_ref, gt_ref, dg_ref, batch_ref, b_ref,
                               gw_ref, gb_ref, gs_ref)
    h_ref[...] = h
    feats_ref[...] = feats


def _tc_norm_last_body(agg_ref, gt_ref, dg_ref, batch_ref, b_ref, gw_ref,
                       gb_ref, gs_ref, f0_ref, f1_ref, merge_ref):
    _, feats = _norm_pool_half(agg_ref, gt_ref, dg_ref, batch_ref, b_ref,
                               gw_ref, gb_ref, gs_ref)
    merge_ref[...] = (f0_ref[...] + f1_ref[...] + feats) * (1.0 / 3.0)


_half_rows = pl.BlockSpec((N, HH), lambda f: (f, 0))
_half_cols_row = pl.BlockSpec((1, HH), lambda f: (0, f))
_half_cols_g = pl.BlockSpec((G, HH), lambda f: (0, f))
_full_deg = pl.BlockSpec((2 * N, 8), lambda f: (0, 0))
_full_batch = pl.BlockSpec((1, N), lambda f: (0, 0))

_tc_norm_mid = pl.pallas_call(
    _tc_norm_mid_body,
    grid=(2,),
    in_specs=[_half_rows, _half_rows, _full_deg, _full_batch,
              _half_cols_row, _half_cols_row, _half_cols_row, _half_cols_row],
    out_specs=[_half_rows, _half_cols_g],
    out_shape=[
        jax.ShapeDtypeStruct((2 * N, HH), jnp.float32),
        jax.ShapeDtypeStruct((G, H), jnp.float32),
    ],
    compiler_params=_TC_PARAMS,
)

_tc_norm_last = pl.pallas_call(
    _tc_norm_last_body,
    grid=(2,),
    in_specs=[_half_rows, _half_rows, _full_deg, _full_batch,
              _half_cols_row, _half_cols_row, _half_cols_row, _half_cols_row,
              _half_cols_g, _half_cols_g],
    out_specs=_half_cols_g,
    out_shape=jax.ShapeDtypeStruct((G, H), jnp.float32),
    compiler_params=_TC_PARAMS,
)


def _tc_matmul_body(h_ref, w_ref, dg_ref, out_ref):
    gn = (_mm3(h_ref[0:N, :], w_ref[0:HH, :])
          + _mm3(h_ref[N:2 * N, :], w_ref[HH:H, :]))
    _store_halves(out_ref, gn * _dinv_from(dg_ref))


_tc_matmul = pl.pallas_call(
    _tc_matmul_body,
    out_shape=jax.ShapeDtypeStruct((GROWS, HH), jnp.float32),
    compiler_params=_TC_PARAMS,
)


def kernel(x, edge_index, batch, W_emb, b_emb, W0, b0, gn_w0, gn_b0, gn_s0,
           W1, b1, gn_w1, gn_b1, gn_s1, W2, b2, gn_w2, gn_b2, gn_s2):
    src = edge_index[0]
    dst = edge_index[1]
    src_t = src.reshape(NS, EPT)
    pad = ((0, 0), (0, EPTP - EPT))
    src0 = jnp.pad(src_t, pad, constant_values=2 * N)
    src1 = jnp.pad(src_t + N, pad, constant_values=2 * N)
    srcp = jnp.concatenate([src0, src1], axis=0).reshape(2 * NS, CT, K)
    dstp = jnp.pad(dst.reshape(NS, EPT), pad,
                   constant_values=DUMMY).reshape(NS, CT, K)

    ones_g = jnp.ones((GROWS, HH), jnp.float32)
    z128 = jnp.zeros((ZR, HH), jnp.float32)
    batch_r = batch.reshape(1, N)
    be = b_emb.reshape(1, H)
    layer = [
        (W0, b0.reshape(1, H), gn_w0.reshape(1, H), gn_b0.reshape(1, H),
         gn_s0.reshape(1, H)),
        (W1, b1.reshape(1, H), gn_w1.reshape(1, H), gn_b1.reshape(1, H),
         gn_s1.reshape(1, H)),
        (W2, b2.reshape(1, H), gn_w2.reshape(1, H), gn_b2.reshape(1, H),
         gn_s2.reshape(1, H)),
    ]

    degacc = _sc_scatter(ones_g, srcp, dstp, z128)[:, 0:8]

    gt0 = _tc_embed(x, W_emb, be, W0, degacc)
    agg0 = _sc_scatter(gt0, srcp, dstp, z128)
    h1, f0 = _tc_norm_mid(agg0, gt0, degacc, batch_r, layer[0][1],
                          layer[0][2], layer[0][3], layer[0][4])
    gt1 = _tc_matmul(h1, W1, degacc)
    agg1 = _sc_scatter(gt1, srcp, dstp, z128)
    h2, f1 = _tc_norm_mid(agg1, gt1, degacc, batch_r, layer[1][1],
                          layer[1][2], layer[1][3], layer[1][4])
    gt2 = _tc_matmul(h2, W2, degacc)
    agg2 = _sc_scatter(gt2, srcp, dstp, z128)
    merge = _tc_norm_last(agg2, gt2, degacc, batch_r, layer[2][1],
                          layer[2][2], layer[2][3], layer[2][4], f0, f1)
    return merge


# speedup vs baseline: 1.2747x; 1.2747x over previous
import functools

import jax
import jax.numpy as jnp
from jax import lax
from jax.experimental import pallas as pl
from jax.experimental.pallas import tpu as pltpu
from jax.experimental.pallas import tpu_sc as plsc

N = 10000
E = 320000
IN_C = 128
H = 256
HH = 128
G = 64
NC = 2
NS = 16
K = 128
EPT = E // NS
IDXG = 32
NGRP = 5
CT = IDXG * NGRP
EPTP = CT * K
GROWS = 2 * N + 8
DUMMY = N
ZR = 632
ACCR = ZR * NS
DPT = 624

_MESH = plsc.VectorSubcoreMesh(core_axis_name="c", subcore_axis_name="s")


def _dump(acc, out_hbm, c, s):
    @pl.when(s < NS - 1)
    def _():
        pltpu.sync_copy(acc.at[pl.ds(s * DPT, DPT)],
                        out_hbm.at[pl.ds(c * N + s * DPT, DPT)])

    @pl.when(s == NS - 1)
    def _():
        pltpu.sync_copy(acc.at[pl.ds((NS - 1) * DPT, N - (NS - 1) * DPT)],
                        out_hbm.at[pl.ds(c * N + (NS - 1) * DPT,
                                         N - (NS - 1) * DPT)])

@functools.partial(
    pl.kernel,
    out_type=jax.ShapeDtypeStruct((2 * N, HH), jnp.float32),
    mesh=_MESH,
    scratch_types=[
        pltpu.VMEM((CT, K), jnp.int32),
        pltpu.VMEM((K, HH), jnp.float32),
        pltpu.VMEM_SHARED((ACCR, HH), jnp.float32),
        pltpu.SemaphoreType.DMA,
    ],
)
def _sc_deg(dstp_hbm, ones_hbm, z_hbm, out_hbm, dst_v, ones_v, acc, ssem):
    c = lax.axis_index("c")
    s = lax.axis_index("s")
    pltpu.sync_copy(z_hbm, acc.at[pl.ds(s * ZR, ZR)])
    pltpu.sync_copy(dstp_hbm.at[s], dst_v)
    pltpu.sync_copy(ones_hbm, ones_v)
    plsc.subcore_barrier()
    half = CT // 2
    lo = c * half

    def fire8(gq, carry):
        base = lo + gq * 8
        for u in range(8):
            pltpu.async_copy(ones_v, acc.at[dst_v.at[base + u]], ssem,
                             add=True)
        for _ in range(8):
            pltpu.make_async_copy(ones_v, acc.at[dst_v.at[base]],
                                  ssem).wait()
        return carry

    lax.fori_loop(0, half // 8, fire8, 0)
    plsc.subcore_barrier()
    _dump(acc, out_hbm, c, s)


@functools.partial(
    pl.kernel,
    out_type=jax.ShapeDtypeStruct((2 * N, HH), jnp.float32),
    mesh=_MESH,
    scratch_types=[
        pltpu.VMEM((IDXG, K), jnp.int32),
        pltpu.VMEM((IDXG, K), jnp.int32),
        pltpu.VMEM((K, HH), jnp.float32),
        pltpu.VMEM((K, HH), jnp.float32),
        pltpu.VMEM_SHARED((ACCR, HH), jnp.float32),
        pltpu.SemaphoreType.DMA,
        pltpu.SemaphoreType.DMA,
        pltpu.SemaphoreType.DMA,
        pltpu.SemaphoreType.DMA,
    ],
)
def _sc_scatter(g_hbm, srcp_hbm, dstp_hbm, z_hbm, out_hbm,
                src_v, dst_v, rows_a, rows_b, acc,
                gsem_a, gsem_b, ssem_a, ssem_b):
    c = lax.axis_index("c")
    s = lax.axis_index("s")
    tid = c * NS + s
    pltpu.sync_copy(z_hbm, acc.at[pl.ds(s * ZR, ZR)])
    plsc.subcore_barrier()

    def group(gi, carry):
        pltpu.sync_copy(srcp_hbm.at[tid, pl.ds(gi * IDXG, IDXG)], src_v)
        pltpu.sync_copy(dstp_hbm.at[s, pl.ds(gi * IDXG, IDXG)], dst_v)
        pltpu.async_copy(g_hbm.at[src_v.at[0]], rows_a, gsem_a)

        def step(j, cur, gsem_cur, ssem_cur, nxt, gsem_nxt, ssem_nxt):
            pltpu.make_async_copy(g_hbm.at[src_v.at[j]], cur,
                                  gsem_cur).wait()

            @pl.when(j + 1 < IDXG)
            def _():
                @pl.when(j >= 1)
                def _():
                    pltpu.make_async_copy(nxt, acc.at[dst_v.at[j]],
                                          ssem_nxt).wait()

                pltpu.async_copy(g_hbm.at[src_v.at[j + 1]], nxt, gsem_nxt)

            pltpu.async_copy(cur, acc.at[dst_v.at[j]], ssem_cur, add=True)

        def pair(k, carry2):
            step(2 * k, rows_a, gsem_a, ssem_a, rows_b, gsem_b, ssem_b)
            step(2 * k + 1, rows_b, gsem_b, ssem_b, rows_a, gsem_a, ssem_a)
            return carry2

        carry = lax.fori_loop(0, IDXG // 2, pair, carry)
        pltpu.make_async_copy(rows_a, acc.at[dst_v.at[0]], ssem_a).wait()
        pltpu.make_async_copy(rows_b, acc.at[dst_v.at[0]], ssem_b).wait()
        return carry

    lax.fori_loop(0, NGRP, group, 0)
    plsc.subcore_barrier()
    _dump(acc, out_hbm, c, s)


_TC_PARAMS = pltpu.CompilerParams(vmem_limit_bytes=60 * 1024 * 1024)

def _dinv_from(dg_ref):
    deg = dg_ref[0:N, 0:1] + dg_ref[N:2 * N, 0:1] + 1.0
    return lax.rsqrt(deg)


def _split_cols(ref):
    return jnp.concatenate([ref[0:N, :], ref[N:2 * N, :]], axis=1)


def _store_halves(out_ref, gn):
    out_ref[0:N, :] = gn[:, 0:HH]
    out_ref[N:2 * N, :] = gn[:, HH:H]
    out_ref[2 * N:GROWS, :] = jnp.zeros((GROWS - 2 * N, HH), jnp.float32)


def _split_bf16(x):
    hi = x.astype(jnp.bfloat16)
    lo = (x - hi.astype(jnp.float32)).astype(jnp.bfloat16)
    return hi, lo


def _mm3(x, w):
    xh, xl = _split_bf16(x)
    wh, wl = _split_bf16(w)
    a = jnp.concatenate([xh, xl, xh], axis=1)
    b = jnp.concatenate([wh, wh, wl], axis=0)
    return jnp.dot(a, b, preferred_element_type=jnp.float32)


def _pdot(p_bf, x):
    xh, xl = _split_bf16(x)
    return (jnp.dot(p_bf, xh, preferred_element_type=jnp.float32)
            + jnp.dot(p_bf, xl, preferred_element_type=jnp.float32))


_BCAST_DIMS = (((0,), (0,)), ((), ()))


def _pbcast(p_bf, x):
    xh, xl = _split_bf16(x)
    return (lax.dot_general(p_bf, xh, _BCAST_DIMS,
                            preferred_element_type=jnp.float32)
            + lax.dot_general(p_bf, xl, _BCAST_DIMS,
                              preferred_element_type=jnp.float32))


def _tc_embed_body(x_ref, we_ref, be_ref, w0_ref, dg_ref, out_ref):
    wc = _mm3(we_ref[...], w0_ref[...])
    bc = _mm3(be_ref[...], w0_ref[...])
    g = _mm3(x_ref[...], wc) + bc
    _store_halves(out_ref, g * _dinv_from(dg_ref))


_tc_embed = pl.pallas_call(
    _tc_embed_body,
    out_shape=jax.ShapeDtypeStruct((GROWS, HH), jnp.float32),
    compiler_params=_TC_PARAMS,
)


def _norm_pool_half(agg_ref, gt_ref, dg_ref, batch_ref, b_ref, gw_ref,
                    gb_ref, gs_ref):
    dinv = _dinv_from(dg_ref)
    t = dinv * (agg_ref[...] + gt_ref[...]) + b_ref[...]
    p = (lax.broadcasted_iota(jnp.int32, (G, N), 0)
         == batch_ref[...]).astype(jnp.bfloat16)
    cnt = jnp.maximum(jnp.sum(p.astype(jnp.float32), axis=1,
                              keepdims=True), 1.0)
    mean = _pdot(p, t) / cnt
    meanfull = _pbcast(p, mean)
    o = t - meanfull * gs_ref[...]
    var = _pdot(p, o * o) / cnt
    inv_std = lax.rsqrt(var + 1e-5)
    isf = _pbcast(p, inv_std)
    h = gw_ref[...] * o * isf + gb_ref[...]
    h = jnp.where(h >= 0, h, 0.01 * h)
    feats = _pdot(p, h) / cnt
    return h, feats


def _tc_norm_mid_body(agg_ref, gt_ref, dg_ref, batch_ref, b_ref, gw_ref,
                      gb_ref, gs_ref, h_ref, feats_ref):
    h, feats = _norm_pool_half(agg_ref, gt_ref, dg_ref, batch_ref, b_ref,
                               gw_ref, gb_ref, gs_ref)
    h_ref[...] = h
    feats_ref[...] = feats


def _tc_norm_last_body(agg_ref, gt_ref, dg_ref, batch_ref, b_ref, gw_ref,
                       gb_ref, gs_ref, f0_ref, f1_ref, merge_ref):
    _, feats = _norm_pool_half(agg_ref, gt_ref, dg_ref, batch_ref, b_ref,
                               gw_ref, gb_ref, gs_ref)
    merge_ref[...] = (f0_ref[...] + f1_ref[...] + feats) * (1.0 / 3.0)


_half_rows = pl.BlockSpec((N, HH), lambda f: (f, 0))
_half_cols_row = pl.BlockSpec((1, HH), lambda f: (0, f))
_half_cols_g = pl.BlockSpec((G, HH), lambda f: (0, f))
_full_deg = pl.BlockSpec((2 * N, 8), lambda f: (0, 0))
_full_batch = pl.BlockSpec((1, N), lambda f: (0, 0))

_tc_norm_mid = pl.pallas_call(
    _tc_norm_mid_body,
    grid=(2,),
    in_specs=[_half_rows, _half_rows, _full_deg, _full_batch,
              _half_cols_row, _half_cols_row, _half_cols_row, _half_cols_row],
    out_specs=[_half_rows, _half_cols_g],
    out_shape=[
        jax.ShapeDtypeStruct((2 * N, HH), jnp.float32),
        jax.ShapeDtypeStruct((G, H), jnp.float32),
    ],
    compiler_params=_TC_PARAMS,
)

_tc_norm_last = pl.pallas_call(
    _tc_norm_last_body,
    grid=(2,),
    in_specs=[_half_rows, _half_rows, _full_deg, _full_batch,
              _half_cols_row, _half_cols_row, _half_cols_row, _half_cols_row,
              _half_cols_g, _half_cols_g],
    out_specs=_half_cols_g,
    out_shape=jax.ShapeDtypeStruct((G, H), jnp.float32),
    compiler_params=_TC_PARAMS,
)


def _tc_matmul_body(h_ref, w_ref, dg_ref, out_ref):
    gn = (_mm3(h_ref[0:N, :], w_ref[0:HH, :])
          + _mm3(h_ref[N:2 * N, :], w_ref[HH:H, :]))
    _store_halves(out_ref, gn * _dinv_from(dg_ref))


_tc_matmul = pl.pallas_call(
    _tc_matmul_body,
    out_shape=jax.ShapeDtypeStruct((GROWS, HH), jnp.float32),
    compiler_params=_TC_PARAMS,
)


def kernel(x, edge_index, batch, W_emb, b_emb, W0, b0, gn_w0, gn_b0, gn_s0,
           W1, b1, gn_w1, gn_b1, gn_s1, W2, b2, gn_w2, gn_b2, gn_s2):
    src = edge_index[0]
    dst = edge_index[1]
    src_t = src.reshape(NS, EPT)
    pad = ((0, 0), (0, EPTP - EPT))
    src0 = jnp.pad(src_t, pad, constant_values=2 * N)
    src1 = jnp.pad(src_t + N, pad, constant_values=2 * N)
    srcp = jnp.concatenate([src0, src1], axis=0).reshape(2 * NS, CT, K)
    dstp = jnp.pad(dst.reshape(NS, EPT), pad,
                   constant_values=DUMMY).reshape(NS, CT, K)

    ones_rows = jnp.ones((K, HH), jnp.float32)
    z128 = jnp.zeros((ZR, HH), jnp.float32)
    batch_r = batch.reshape(1, N)
    be = b_emb.reshape(1, H)
    layer = [
        (W0, b0.reshape(1, H), gn_w0.reshape(1, H), gn_b0.reshape(1, H),
         gn_s0.reshape(1, H)),
        (W1, b1.reshape(1, H), gn_w1.reshape(1, H), gn_b1.reshape(1, H),
         gn_s1.reshape(1, H)),
        (W2, b2.reshape(1, H), gn_w2.reshape(1, H), gn_b2.reshape(1, H),
         gn_s2.reshape(1, H)),
    ]

    degacc = _sc_deg(dstp, ones_rows, z128)[:, 0:8]

    gt0 = _tc_embed(x, W_emb, be, W0, degacc)
    agg0 = _sc_scatter(gt0, srcp, dstp, z128)
    h1, f0 = _tc_norm_mid(agg0, gt0, degacc, batch_r, layer[0][1],
                          layer[0][2], layer[0][3], layer[0][4])
    gt1 = _tc_matmul(h1, W1, degacc)
    agg1 = _sc_scatter(gt1, srcp, dstp, z128)
    h2, f1 = _tc_norm_mid(agg1, gt1, degacc, batch_r, layer[1][1],
                          layer[1][2], layer[1][3], layer[1][4])
    gt2 = _tc_matmul(h2, W2, degacc)
    agg2 = _sc_scatter(gt2, srcp, dstp, z128)
    merge = _tc_norm_last(agg2, gt2, degacc, batch_r, layer[2][1],
                          layer[2][2], layer[2][3], layer[2][4], f0, f1)
    return merge


# speedup vs baseline: 1.3332x; 1.0459x over previous
import functools

import jax
import jax.numpy as jnp
from jax import lax
from jax.experimental import pallas as pl
from jax.experimental.pallas import tpu as pltpu
from jax.experimental.pallas import tpu_sc as plsc

N = 10000
E = 320000
IN_C = 128
H = 256
HH = 128
G = 64
NC = 2
NS = 16
K = 128
EPT = E // NS
IDXG = 32
NGRP = 5
CT = IDXG * NGRP
EPTP = CT * K
GROWS = 2 * N + 8
DUMMY = N
ZR = 632
ACCR = ZR * NS
DPT = 624

_MESH = plsc.VectorSubcoreMesh(core_axis_name="c", subcore_axis_name="s")


def _dump(acc, out_hbm, c, s):
    @pl.when(s < NS - 1)
    def _():
        pltpu.sync_copy(acc.at[pl.ds(s * DPT, DPT)],
                        out_hbm.at[pl.ds(c * N + s * DPT, DPT)])

    @pl.when(s == NS - 1)
    def _():
        pltpu.sync_copy(acc.at[pl.ds((NS - 1) * DPT, N - (NS - 1) * DPT)],
                        out_hbm.at[pl.ds(c * N + (NS - 1) * DPT,
                                         N - (NS - 1) * DPT)])

@functools.partial(
    pl.kernel,
    out_type=jax.ShapeDtypeStruct((2 * N, HH), jnp.float32),
    mesh=_MESH,
    scratch_types=[
        pltpu.VMEM((CT, K), jnp.int32),
        pltpu.VMEM((K, HH), jnp.float32),
        pltpu.VMEM_SHARED((ACCR, HH), jnp.float32),
        pltpu.SemaphoreType.DMA,
    ],
)
def _sc_deg(dstp_hbm, ones_hbm, z_hbm, out_hbm, dst_v, ones_v, acc, ssem):
    c = lax.axis_index("c")
    s = lax.axis_index("s")
    pltpu.sync_copy(z_hbm, acc.at[pl.ds(s * ZR, ZR)])
    pltpu.sync_copy(dstp_hbm.at[s], dst_v)
    pltpu.sync_copy(ones_hbm, ones_v)
    plsc.subcore_barrier()
    half = CT // 2
    lo = c * half

    def fire8(gq, carry):
        base = lo + gq * 8
        for u in range(8):
            pltpu.async_copy(ones_v, acc.at[dst_v.at[base + u]], ssem,
                             add=True)
        for _ in range(8):
            pltpu.make_async_copy(ones_v, acc.at[dst_v.at[base]],
                                  ssem).wait()
        return carry

    lax.fori_loop(0, half // 8, fire8, 0)
    plsc.subcore_barrier()
    _dump(acc, out_hbm, c, s)


@functools.partial(
    pl.kernel,
    out_type=jax.ShapeDtypeStruct((2 * N, HH), jnp.float32),
    mesh=_MESH,
    scratch_types=[
        pltpu.VMEM((IDXG, K), jnp.int32),
        pltpu.VMEM((IDXG, K), jnp.int32),
        pltpu.VMEM((K, HH), jnp.float32),
        pltpu.VMEM((K, HH), jnp.float32),
        pltpu.VMEM_SHARED((ACCR, HH), jnp.float32),
        pltpu.SemaphoreType.DMA,
        pltpu.SemaphoreType.DMA,
        pltpu.SemaphoreType.DMA,
        pltpu.SemaphoreType.DMA,
    ],
)
def _sc_scatter(g_hbm, srcp_hbm, dstp_hbm, z_hbm, out_hbm,
                src_v, dst_v, rows_a, rows_b, acc,
                gsem_a, gsem_b, ssem_a, ssem_b):
    c = lax.axis_index("c")
    s = lax.axis_index("s")
    tid = c * NS + s
    pltpu.sync_copy(z_hbm, acc.at[pl.ds(s * ZR, ZR)])
    plsc.subcore_barrier()

    def group(gi, carry):
        pltpu.sync_copy(srcp_hbm.at[tid, pl.ds(gi * IDXG, IDXG)], src_v)
        pltpu.sync_copy(dstp_hbm.at[s, pl.ds(gi * IDXG, IDXG)], dst_v)
        pltpu.async_copy(g_hbm.at[src_v.at[0]], rows_a, gsem_a)

        def step(j, cur, gsem_cur, ssem_cur, nxt, gsem_nxt, ssem_nxt):
            @pl.when(j + 1 < IDXG)
            def _():
                @pl.when(j >= 1)
                def _():
                    pltpu.make_async_copy(nxt, acc.at[dst_v.at[j]],
                                          ssem_nxt).wait()

                pltpu.async_copy(g_hbm.at[src_v.at[j + 1]], nxt, gsem_nxt)

            pltpu.make_async_copy(g_hbm.at[src_v.at[j]], cur,
                                  gsem_cur).wait()
            pltpu.async_copy(cur, acc.at[dst_v.at[j]], ssem_cur, add=True)

        def pair(k, carry2):
            step(2 * k, rows_a, gsem_a, ssem_a, rows_b, gsem_b, ssem_b)
            step(2 * k + 1, rows_b, gsem_b, ssem_b, rows_a, gsem_a, ssem_a)
            return carry2

        carry = lax.fori_loop(0, IDXG // 2, pair, carry)
        pltpu.make_async_copy(rows_a, acc.at[dst_v.at[0]], ssem_a).wait()
        pltpu.make_async_copy(rows_b, acc.at[dst_v.at[0]], ssem_b).wait()
        return carry

    lax.fori_loop(0, NGRP, group, 0)
    plsc.subcore_barrier()
    _dump(acc, out_hbm, c, s)


_TC_PARAMS = pltpu.CompilerParams(vmem_limit_bytes=60 * 1024 * 1024)

def _dinv_from(dg_ref):
    deg = dg_ref[0:N, 0:1] + dg_ref[N:2 * N, 0:1] + 1.0
    return lax.rsqrt(deg)


def _split_cols(ref):
    return jnp.concatenate([ref[0:N, :], ref[N:2 * N, :]], axis=1)


def _store_halves(out_ref, gn):
    out_ref[0:N, :] = gn[:, 0:HH]
    out_ref[N:2 * N, :] = gn[:, HH:H]
    out_ref[2 * N:GROWS, :] = jnp.zeros((GROWS - 2 * N, HH), jnp.float32)


def _split_bf16(x):
    hi = x.astype(jnp.bfloat16)
    lo = (x - hi.astype(jnp.float32)).astype(jnp.bfloat16)
    return hi, lo


def _mm3(x, w):
    xh, xl = _split_bf16(x)
    wh, wl = _split_bf16(w)
    a = jnp.concatenate([xh, xl, xh], axis=1)
    b = jnp.concatenate([wh, wh, wl], axis=0)
    return jnp.dot(a, b, preferred_element_type=jnp.float32)


def _pdot(p_bf, x):
    xh, xl = _split_bf16(x)
    return (jnp.dot(p_bf, xh, preferred_element_type=jnp.float32)
            + jnp.dot(p_bf, xl, preferred_element_type=jnp.float32))


_BCAST_DIMS = (((0,), (0,)), ((), ()))


def _pbcast(p_bf, x):
    xh, xl = _split_bf16(x)
    return (lax.dot_general(p_bf, xh, _BCAST_DIMS,
                            preferred_element_type=jnp.float32)
            + lax.dot_general(p_bf, xl, _BCAST_DIMS,
                              preferred_element_type=jnp.float32))


def _tc_embed_body(x_ref, we_ref, be_ref, w0_ref, dg_ref, out_ref):
    wc = _mm3(we_ref[...], w0_ref[...])
    bc = _mm3(be_ref[...], w0_ref[...])
    g = _mm3(x_ref[...], wc) + bc
    _store_halves(out_ref, g * _dinv_from(dg_ref))


_tc_embed = pl.pallas_call(
    _tc_embed_body,
    out_shape=jax.ShapeDtypeStruct((GROWS, HH), jnp.float32),
    compiler_params=_TC_PARAMS,
)


def _norm_pool_half(agg_ref, gt_ref, dg_ref, batch_ref, b_ref, gw_ref,
                    gb_ref, gs_ref):
    dinv = _dinv_from(dg_ref)
    t = dinv * (agg_ref[...] + gt_ref[...]) + b_ref[...]
    p = (lax.broadcasted_iota(jnp.int32, (G, N), 0)
         == batch_ref[...]).astype(jnp.bfloat16)
    cnt = jnp.maximum(jnp.sum(p.astype(jnp.float32), axis=1,
                              keepdims=True), 1.0)
    mean = _pdot(p, t) / cnt
    meanfull = _pbcast(p, mean)
    o = t - meanfull * gs_ref[...]
    var = _pdot(p, o * o) / cnt
    inv_std = lax.rsqrt(var + 1e-5)
    isf = _pbcast(p, inv_std)
    h = gw_ref[...] * o * isf + gb_ref[...]
    h = jnp.where(h >= 0, h, 0.01 * h)
    feats = _pdot(p, h) / cnt
    return h, feats


def _tc_norm_mid_body(agg_ref, gt_ref, dg_ref, batch_ref, b_ref, gw_ref,
                      gb_ref, gs_ref, h_ref, feats_ref):
    h, feats = _norm_pool_half(agg_ref, gt_ref, dg_ref, batch_ref, b_ref,
                               gw_ref, gb_ref, gs_ref)
    h_ref[...] = h
    feats_ref[...] = feats


def _tc_norm_last_body(agg_ref, gt_ref, dg_ref, batch_ref, b_ref, gw_ref,
                       gb_ref, gs_ref, f0_ref, f1_ref, merge_ref):
    _, feats = _norm_pool_half(agg_ref, gt_ref, dg_ref, batch_ref, b_ref,
                               gw_ref, gb_ref, gs_ref)
    merge_ref[...] = (f0_ref[...] + f1_ref[...] + feats) * (1.0 / 3.0)


_half_rows = pl.BlockSpec((N, HH), lambda f: (f, 0))
_half_cols_row = pl.BlockSpec((1, HH), lambda f: (0, f))
_half_cols_g = pl.BlockSpec((G, HH), lambda f: (0, f))
_full_deg = pl.BlockSpec((2 * N, 8), lambda f: (0, 0))
_full_batch = pl.BlockSpec((1, N), lambda f: (0, 0))

_tc_norm_mid = pl.pallas_call(
    _tc_norm_mid_body,
    grid=(2,),
    in_specs=[_half_rows, _half_rows, _full_deg, _full_batch,
              _half_cols_row, _half_cols_row, _half_cols_row, _half_cols_row],
    out_specs=[_half_rows, _half_cols_g],
    out_shape=[
        jax.ShapeDtypeStruct((2 * N, HH), jnp.float32),
        jax.ShapeDtypeStruct((G, H), jnp.float32),
    ],
    compiler_params=_TC_PARAMS,
)

_tc_norm_last = pl.pallas_call(
    _tc_norm_last_body,
    grid=(2,),
    in_specs=[_half_rows, _half_rows, _full_deg, _full_batch,
              _half_cols_row, _half_cols_row, _half_cols_row, _half_cols_row,
              _half_cols_g, _half_cols_g],
    out_specs=_half_cols_g,
    out_shape=jax.ShapeDtypeStruct((G, H), jnp.float32),
    compiler_params=_TC_PARAMS,
)


def _tc_matmul_body(h_ref, w_ref, dg_ref, out_ref):
    gn = (_mm3(h_ref[0:N, :], w_ref[0:HH, :])
          + _mm3(h_ref[N:2 * N, :], w_ref[HH:H, :]))
    _store_halves(out_ref, gn * _dinv_from(dg_ref))


_tc_matmul = pl.pallas_call(
    _tc_matmul_body,
    out_shape=jax.ShapeDtypeStruct((GROWS, HH), jnp.float32),
    compiler_params=_TC_PARAMS,
)


def kernel(x, edge_index, batch, W_emb, b_emb, W0, b0, gn_w0, gn_b0, gn_s0,
           W1, b1, gn_w1, gn_b1, gn_s1, W2, b2, gn_w2, gn_b2, gn_s2):
    src = edge_index[0]
    dst = edge_index[1]
    src_t = src.reshape(NS, EPT)
    pad = ((0, 0), (0, EPTP - EPT))
    src0 = jnp.pad(src_t, pad, constant_values=2 * N)
    src1 = jnp.pad(src_t + N, pad, constant_values=2 * N)
    srcp = jnp.concatenate([src0, src1], axis=0).reshape(2 * NS, CT, K)
    dstp = jnp.pad(dst.reshape(NS, EPT), pad,
                   constant_values=DUMMY).reshape(NS, CT, K)

    ones_rows = jnp.ones((K, HH), jnp.float32)
    z128 = jnp.zeros((ZR, HH), jnp.float32)
    batch_r = batch.reshape(1, N)
    be = b_emb.reshape(1, H)
    layer = [
        (W0, b0.reshape(1, H), gn_w0.reshape(1, H), gn_b0.reshape(1, H),
         gn_s0.reshape(1, H)),
        (W1, b1.reshape(1, H), gn_w1.reshape(1, H), gn_b1.reshape(1, H),
         gn_s1.reshape(1, H)),
        (W2, b2.reshape(1, H), gn_w2.reshape(1, H), gn_b2.reshape(1, H),
         gn_s2.reshape(1, H)),
    ]

    degacc = _sc_deg(dstp, ones_rows, z128)[:, 0:8]

    gt0 = _tc_embed(x, W_emb, be, W0, degacc)
    agg0 = _sc_scatter(gt0, srcp, dstp, z128)
    h1, f0 = _tc_norm_mid(agg0, gt0, degacc, batch_r, layer[0][1],
                          layer[0][2], layer[0][3], layer[0][4])
    gt1 = _tc_matmul(h1, W1, degacc)
    agg1 = _sc_scatter(gt1, srcp, dstp, z128)
    h2, f1 = _tc_norm_mid(agg1, gt1, degacc, batch_r, layer[1][1],
                          layer[1][2], layer[1][3], layer[1][4])
    gt2 = _tc_matmul(h2, W2, degacc)
    agg2 = _sc_scatter(gt2, srcp, dstp, z128)
    merge = _tc_norm_last(agg2, gt2, degacc, batch_r, layer[2][1],
                          layer[2][2], layer[2][3], layer[2][4], f0, f1)
    return merge


# speedup vs baseline: 1.3334x; 1.0002x over previous
import functools

import jax
import jax.numpy as jnp
from jax import lax
from jax.experimental import pallas as pl
from jax.experimental.pallas import tpu as pltpu
from jax.experimental.pallas import tpu_sc as plsc

N = 10000
E = 320000
IN_C = 128
H = 256
HH = 128
G = 64
NC = 2
NS = 16
K = 128
EPT = E // NS
IDXG = 32
NGRP = 5
CT = IDXG * NGRP
EPTP = CT * K
GROWS = 2 * N + 8
DUMMY = N
ZR = 632
ACCR = ZR * NS
DPT = 624

_MESH = plsc.VectorSubcoreMesh(core_axis_name="c", subcore_axis_name="s")


def _dump(acc, out_hbm, c, s):
    @pl.when(s < NS - 1)
    def _():
        pltpu.sync_copy(acc.at[pl.ds(s * DPT, DPT)],
                        out_hbm.at[pl.ds(c * N + s * DPT, DPT)])

    @pl.when(s == NS - 1)
    def _():
        pltpu.sync_copy(acc.at[pl.ds((NS - 1) * DPT, N - (NS - 1) * DPT)],
                        out_hbm.at[pl.ds(c * N + (NS - 1) * DPT,
                                         N - (NS - 1) * DPT)])

@functools.partial(
    pl.kernel,
    out_type=jax.ShapeDtypeStruct((2 * N, HH), jnp.float32),
    mesh=_MESH,
    scratch_types=[
        pltpu.VMEM((CT, K), jnp.int32),
        pltpu.VMEM((K, HH), jnp.float32),
        pltpu.VMEM_SHARED((ACCR, HH), jnp.float32),
        pltpu.SemaphoreType.DMA,
    ],
)
def _sc_deg(dstp_hbm, ones_hbm, z_hbm, out_hbm, dst_v, ones_v, acc, ssem):
    c = lax.axis_index("c")
    s = lax.axis_index("s")
    pltpu.sync_copy(z_hbm, acc.at[pl.ds(s * ZR, ZR)])
    pltpu.sync_copy(dstp_hbm.at[s], dst_v)
    pltpu.sync_copy(ones_hbm, ones_v)
    plsc.subcore_barrier()
    half = CT // 2
    lo = c * half

    def fire8(gq, carry):
        base = lo + gq * 8
        for u in range(8):
            pltpu.async_copy(ones_v, acc.at[dst_v.at[base + u]], ssem,
                             add=True)
        for _ in range(8):
            pltpu.make_async_copy(ones_v, acc.at[dst_v.at[base]],
                                  ssem).wait()
        return carry

    lax.fori_loop(0, half // 8, fire8, 0)
    plsc.subcore_barrier()
    _dump(acc, out_hbm, c, s)


@functools.partial(
    pl.kernel,
    out_type=jax.ShapeDtypeStruct((2 * N, HH), jnp.float32),
    mesh=_MESH,
    scratch_types=[
        pltpu.VMEM((IDXG, K), jnp.int32),
        pltpu.VMEM((IDXG, K), jnp.int32),
        pltpu.VMEM((K, HH), jnp.float32),
        pltpu.VMEM((K, HH), jnp.float32),
        pltpu.VMEM_SHARED((ACCR, HH), jnp.float32),
        pltpu.SemaphoreType.DMA,
        pltpu.SemaphoreType.DMA,
        pltpu.SemaphoreType.DMA,
        pltpu.SemaphoreType.DMA,
    ],
)
def _sc_scatter(g_hbm, srcp_hbm, dstp_hbm, z_hbm, out_hbm,
                src_v, dst_v, rows_a, rows_b, acc,
                gsem_a, gsem_b, ssem_a, ssem_b):
    c = lax.axis_index("c")
    s = lax.axis_index("s")
    tid = c * NS + s
    pltpu.sync_copy(z_hbm, acc.at[pl.ds(s * ZR, ZR)])
    plsc.subcore_barrier()

    def group(gi, carry):
        pltpu.sync_copy(srcp_hbm.at[tid, pl.ds(gi * IDXG, IDXG)], src_v)
        pltpu.sync_copy(dstp_hbm.at[s, pl.ds(gi * IDXG, IDXG)], dst_v)
        def gather2(j, buf, gsem):
            pltpu.async_copy(g_hbm.at[src_v.at[j, pl.ds(0, K // 2)]],
                             buf.at[pl.ds(0, K // 2)], gsem)
            pltpu.async_copy(g_hbm.at[src_v.at[j, pl.ds(K // 2, K // 2)]],
                             buf.at[pl.ds(K // 2, K // 2)], gsem)

        gather2(0, rows_a, gsem_a)

        def step(j, cur, gsem_cur, ssem_cur, nxt, gsem_nxt, ssem_nxt):
            @pl.when(j + 1 < IDXG)
            def _():
                @pl.when(j >= 1)
                def _():
                    pltpu.make_async_copy(nxt, acc.at[dst_v.at[j]],
                                          ssem_nxt).wait()

                gather2(j + 1, nxt, gsem_nxt)

            pltpu.make_async_copy(g_hbm.at[src_v.at[j]], cur,
                                  gsem_cur).wait()
            pltpu.async_copy(cur, acc.at[dst_v.at[j]], ssem_cur, add=True)

        def pair(k, carry2):
            step(2 * k, rows_a, gsem_a, ssem_a, rows_b, gsem_b, ssem_b)
            step(2 * k + 1, rows_b, gsem_b, ssem_b, rows_a, gsem_a, ssem_a)
            return carry2

        carry = lax.fori_loop(0, IDXG // 2, pair, carry)
        pltpu.make_async_copy(rows_a, acc.at[dst_v.at[0]], ssem_a).wait()
        pltpu.make_async_copy(rows_b, acc.at[dst_v.at[0]], ssem_b).wait()
        return carry

    lax.fori_loop(0, NGRP, group, 0)
    plsc.subcore_barrier()
    _dump(acc, out_hbm, c, s)


_TC_PARAMS = pltpu.CompilerParams(vmem_limit_bytes=60 * 1024 * 1024)

def _dinv_from(dg_ref):
    deg = dg_ref[0:N, 0:1] + dg_ref[N:2 * N, 0:1] + 1.0
    return lax.rsqrt(deg)


def _split_cols(ref):
    return jnp.concatenate([ref[0:N, :], ref[N:2 * N, :]], axis=1)


def _store_halves(out_ref, gn):
    out_ref[0:N, :] = gn[:, 0:HH]
    out_ref[N:2 * N, :] = gn[:, HH:H]
    out_ref[2 * N:GROWS, :] = jnp.zeros((GROWS - 2 * N, HH), jnp.float32)


def _split_bf16(x):
    hi = x.astype(jnp.bfloat16)
    lo = (x - hi.astype(jnp.float32)).astype(jnp.bfloat16)
    return hi, lo


def _mm3(x, w):
    xh, xl = _split_bf16(x)
    wh, wl = _split_bf16(w)
    a = jnp.concatenate([xh, xl, xh], axis=1)
    b = jnp.concatenate([wh, wh, wl], axis=0)
    return jnp.dot(a, b, preferred_element_type=jnp.float32)


def _pdot(p_bf, x):
    xh, xl = _split_bf16(x)
    return (jnp.dot(p_bf, xh, preferred_element_type=jnp.float32)
            + jnp.dot(p_bf, xl, preferred_element_type=jnp.float32))


_BCAST_DIMS = (((0,), (0,)), ((), ()))


def _pbcast(p_bf, x):
    xh, xl = _split_bf16(x)
    return (lax.dot_general(p_bf, xh, _BCAST_DIMS,
                            preferred_element_type=jnp.float32)
            + lax.dot_general(p_bf, xl, _BCAST_DIMS,
                              preferred_element_type=jnp.float32))


def _tc_embed_body(x_ref, we_ref, be_ref, w0_ref, dg_ref, out_ref):
    wc = _mm3(we_ref[...], w0_ref[...])
    bc = _mm3(be_ref[...], w0_ref[...])
    g = _mm3(x_ref[...], wc) + bc
    _store_halves(out_ref, g * _dinv_from(dg_ref))


_tc_embed = pl.pallas_call(
    _tc_embed_body,
    out_shape=jax.ShapeDtypeStruct((GROWS, HH), jnp.float32),
    compiler_params=_TC_PARAMS,
)


def _norm_pool_half(agg_ref, gt_ref, dg_ref, batch_ref, b_ref, gw_ref,
                    gb_ref, gs_ref):
    dinv = _dinv_from(dg_ref)
    t = dinv * (agg_ref[...] + gt_ref[...]) + b_ref[...]
    p = (lax.broadcasted_iota(jnp.int32, (G, N), 0)
         == batch_ref[...]).astype(jnp.bfloat16)
    cnt = jnp.maximum(jnp.sum(p.astype(jnp.float32), axis=1,
                              keepdims=True), 1.0)
    mean = _pdot(p, t) / cnt
    meanfull = _pbcast(p, mean)
    o = t - meanfull * gs_ref[...]
    var = _pdot(p, o * o) / cnt
    inv_std = lax.rsqrt(var + 1e-5)
    isf = _pbcast(p, inv_std)
    h = gw_ref[...] * o * isf + gb_ref[...]
    h = jnp.where(h >= 0, h, 0.01 * h)
    feats = _pdot(p, h) / cnt
    return h, feats


def _tc_norm_mid_body(agg_ref, gt_ref, dg_ref, batch_ref, b_ref, gw_ref,
                      gb_ref, gs_ref, h_ref, feats_ref):
    h, feats = _norm_pool_half(agg_ref, gt_ref, dg_ref, batch_ref, b_ref,
                               gw_ref, gb_ref, gs_ref)
    h_ref[...] = h
    feats_ref[...] = feats


def _tc_norm_last_body(agg_ref, gt_ref, dg_ref, batch_ref, b_ref, gw_ref,
                       gb_ref, gs_ref, f0_ref, f1_ref, merge_ref):
    _, feats = _norm_pool_half(agg_ref, gt_ref, dg_ref, batch_ref, b_ref,
                               gw_ref, gb_ref, gs_ref)
    merge_ref[...] = (f0_ref[...] + f1_ref[...] + feats) * (1.0 / 3.0)


_half_rows = pl.BlockSpec((N, HH), lambda f: (f, 0))
_half_cols_row = pl.BlockSpec((1, HH), lambda f: (0, f))
_half_cols_g = pl.BlockSpec((G, HH), lambda f: (0, f))
_full_deg = pl.BlockSpec((2 * N, 8), lambda f: (0, 0))
_full_batch = pl.BlockSpec((1, N), lambda f: (0, 0))

_tc_norm_mid = pl.pallas_call(
    _tc_norm_mid_body,
    grid=(2,),
    in_specs=[_half_rows, _half_rows, _full_deg, _full_batch,
              _half_cols_row, _half_cols_row, _half_cols_row, _half_cols_row],
    out_specs=[_half_rows, _half_cols_g],
    out_shape=[
        jax.ShapeDtypeStruct((2 * N, HH), jnp.float32),
        jax.ShapeDtypeStruct((G, H), jnp.float32),
    ],
    compiler_params=_TC_PARAMS,
)

_tc_norm_last = pl.pallas_call(
    _tc_norm_last_body,
    grid=(2,),
    in_specs=[_half_rows, _half_rows, _full_deg, _full_batch,
              _half_cols_row, _half_cols_row, _half_cols_row, _half_cols_row,
              _half_cols_g, _half_cols_g],
    out_specs=_half_cols_g,
    out_shape=jax.ShapeDtypeStruct((G, H), jnp.float32),
    compiler_params=_TC_PARAMS,
)


def _tc_matmul_body(h_ref, w_ref, dg_ref, out_ref):
    gn = (_mm3(h_ref[0:N, :], w_ref[0:HH, :])
          + _mm3(h_ref[N:2 * N, :], w_ref[HH:H, :]))
    _store_halves(out_ref, gn * _dinv_from(dg_ref))


_tc_matmul = pl.pallas_call(
    _tc_matmul_body,
    out_shape=jax.ShapeDtypeStruct((GROWS, HH), jnp.float32),
    compiler_params=_TC_PARAMS,
)


def kernel(x, edge_index, batch, W_emb, b_emb, W0, b0, gn_w0, gn_b0, gn_s0,
           W1, b1, gn_w1, gn_b1, gn_s1, W2, b2, gn_w2, gn_b2, gn_s2):
    src = edge_index[0]
    dst = edge_index[1]
    src_t = src.reshape(NS, EPT)
    pad = ((0, 0), (0, EPTP - EPT))
    src0 = jnp.pad(src_t, pad, constant_values=2 * N)
    src1 = jnp.pad(src_t + N, pad, constant_values=2 * N)
    srcp = jnp.concatenate([src0, src1], axis=0).reshape(2 * NS, CT, K)
    dstp = jnp.pad(dst.reshape(NS, EPT), pad,
                   constant_values=DUMMY).reshape(NS, CT, K)

    ones_rows = jnp.ones((K, HH), jnp.float32)
    z128 = jnp.zeros((ZR, HH), jnp.float32)
    batch_r = batch.reshape(1, N)
    be = b_emb.reshape(1, H)
    layer = [
        (W0, b0.reshape(1, H), gn_w0.reshape(1, H), gn_b0.reshape(1, H),
         gn_s0.reshape(1, H)),
        (W1, b1.reshape(1, H), gn_w1.reshape(1, H), gn_b1.reshape(1, H),
         gn_s1.reshape(1, H)),
        (W2, b2.reshape(1, H), gn_w2.reshape(1, H), gn_b2.reshape(1, H),
         gn_s2.reshape(1, H)),
    ]

    degacc = _sc_deg(dstp, ones_rows, z128)[:, 0:8]

    gt0 = _tc_embed(x, W_emb, be, W0, degacc)
    agg0 = _sc_scatter(gt0, srcp, dstp, z128)
    h1, f0 = _tc_norm_mid(agg0, gt0, degacc, batch_r, layer[0][1],
                          layer[0][2], layer[0][3], layer[0][4])
    gt1 = _tc_matmul(h1, W1, degacc)
    agg1 = _sc_scatter(gt1, srcp, dstp, z128)
    h2, f1 = _tc_norm_mid(agg1, gt1, degacc, batch_r, layer[1][1],
                          layer[1][2], layer[1][3], layer[1][4])
    gt2 = _tc_matmul(h2, W2, degacc)
    agg2 = _sc_scatter(gt2, srcp, dstp, z128)
    merge = _tc_norm_last(agg2, gt2, degacc, batch_r, layer[2][1],
                          layer[2][2], layer[2][3], layer[2][4], f0, f1)
    return merge


# speedup vs baseline: 1.3344x; 1.0008x over previous
import functools

import jax
import jax.numpy as jnp
from jax import lax
from jax.experimental import pallas as pl
from jax.experimental.pallas import tpu as pltpu
from jax.experimental.pallas import tpu_sc as plsc

N = 10000
E = 320000
IN_C = 128
H = 256
HH = 128
G = 64
NC = 2
NS = 16
K = 128
EPT = E // NS
IDXG = 32
NGRP = 5
CT = IDXG * NGRP
EPTP = CT * K
GROWS = 2 * N + 8
DUMMY = N
ZR = 632
ACCR = ZR * NS
DPT = 624

_MESH = plsc.VectorSubcoreMesh(core_axis_name="c", subcore_axis_name="s",
                               num_cores=NC, num_subcores=NS)


def _dump(acc, out_hbm, c, s):
    @pl.when(s < NS - 1)
    def _():
        pltpu.sync_copy(acc.at[pl.ds(s * DPT, DPT)],
                        out_hbm.at[pl.ds(c * N + s * DPT, DPT)])

    @pl.when(s == NS - 1)
    def _():
        pltpu.sync_copy(acc.at[pl.ds((NS - 1) * DPT, N - (NS - 1) * DPT)],
                        out_hbm.at[pl.ds(c * N + (NS - 1) * DPT,
                                         N - (NS - 1) * DPT)])

@functools.partial(
    pl.kernel,
    out_type=jax.ShapeDtypeStruct((2 * N, HH), jnp.float32),
    mesh=_MESH,
    scratch_types=[
        pltpu.VMEM((CT, K), jnp.int32),
        pltpu.VMEM((K, HH), jnp.float32),
        pltpu.VMEM_SHARED((ACCR, HH), jnp.float32),
        pltpu.SemaphoreType.DMA,
    ],
)
def _sc_deg(dstp_hbm, ones_hbm, z_hbm, out_hbm, dst_v, ones_v, acc, ssem):
    c = lax.axis_index("c")
    s = lax.axis_index("s")
    pltpu.sync_copy(z_hbm, acc.at[pl.ds(s * ZR, ZR)])
    pltpu.sync_copy(dstp_hbm.at[s], dst_v)
    pltpu.sync_copy(ones_hbm, ones_v)
    plsc.subcore_barrier()
    half = CT // 2
    lo = c * half

    def fire8(gq, carry):
        base = lo + gq * 8
        for u in range(8):
            pltpu.async_copy(ones_v, acc.at[dst_v.at[base + u]], ssem,
                             add=True)
        for _ in range(8):
            pltpu.make_async_copy(ones_v, acc.at[dst_v.at[base]],
                                  ssem).wait()
        return carry

    lax.fori_loop(0, half // 8, fire8, 0)
    plsc.subcore_barrier()
    _dump(acc, out_hbm, c, s)


@functools.partial(
    pl.kernel,
    out_type=jax.ShapeDtypeStruct((2 * N, HH), jnp.float32),
    mesh=_MESH,
    scratch_types=[
        pltpu.VMEM((IDXG, K), jnp.int32),
        pltpu.VMEM((IDXG, K), jnp.int32),
        pltpu.VMEM((K, HH), jnp.float32),
        pltpu.VMEM((K, HH), jnp.float32),
        pltpu.VMEM_SHARED((ACCR, HH), jnp.float32),
        pltpu.SemaphoreType.DMA,
        pltpu.SemaphoreType.DMA,
        pltpu.SemaphoreType.DMA,
        pltpu.SemaphoreType.DMA,
    ],
)
def _sc_scatter(g_hbm, srcp_hbm, dstp_hbm, z_hbm, out_hbm,
                src_v, dst_v, rows_a, rows_b, acc,
                gsem_a, gsem_b, ssem_a, ssem_b):
    c = lax.axis_index("c")
    s = lax.axis_index("s")
    tid = c * NS + s
    pltpu.sync_copy(z_hbm, acc.at[pl.ds(s * ZR, ZR)])
    plsc.subcore_barrier()

    def group(gi, carry):
        pltpu.sync_copy(srcp_hbm.at[tid, pl.ds(gi * IDXG, IDXG)], src_v)
        pltpu.sync_copy(dstp_hbm.at[s, pl.ds(gi * IDXG, IDXG)], dst_v)
        def gather2(j, buf, gsem):
            pltpu.async_copy(g_hbm.at[src_v.at[j, pl.ds(0, K // 2)]],
                             buf.at[pl.ds(0, K // 2)], gsem)
            pltpu.async_copy(g_hbm.at[src_v.at[j, pl.ds(K // 2, K // 2)]],
                             buf.at[pl.ds(K // 2, K // 2)], gsem)

        gather2(0, rows_a, gsem_a)

        def step(j, cur, gsem_cur, ssem_cur, nxt, gsem_nxt, ssem_nxt):
            @pl.when(j + 1 < IDXG)
            def _():
                @pl.when(j >= 1)
                def _():
                    pltpu.make_async_copy(nxt, acc.at[dst_v.at[j]],
                                          ssem_nxt).wait()

                gather2(j + 1, nxt, gsem_nxt)

            pltpu.make_async_copy(g_hbm.at[src_v.at[j]], cur,
                                  gsem_cur).wait()
            pltpu.async_copy(cur, acc.at[dst_v.at[j]], ssem_cur, add=True)

        def pair(k, carry2):
            step(2 * k, rows_a, gsem_a, ssem_a, rows_b, gsem_b, ssem_b)
            step(2 * k + 1, rows_b, gsem_b, ssem_b, rows_a, gsem_a, ssem_a)
            return carry2

        carry = lax.fori_loop(0, IDXG // 2, pair, carry)
        pltpu.make_async_copy(rows_a, acc.at[dst_v.at[0]], ssem_a).wait()
        pltpu.make_async_copy(rows_b, acc.at[dst_v.at[0]], ssem_b).wait()
        return carry

    lax.fori_loop(0, NGRP, group, 0)
    plsc.subcore_barrier()
    _dump(acc, out_hbm, c, s)


_TC_PARAMS = pltpu.CompilerParams(vmem_limit_bytes=60 * 1024 * 1024)

def _dinv_from(dg_ref):
    deg = dg_ref[0:N, 0:1] + dg_ref[N:2 * N, 0:1] + 1.0
    return lax.rsqrt(deg)


def _store_halves(out_ref, gn):
    out_ref[0:N, :] = gn[:, 0:HH]
    out_ref[N:2 * N, :] = gn[:, HH:H]
    out_ref[2 * N:GROWS, :] = jnp.zeros((GROWS - 2 * N, HH), jnp.float32)


def _split_bf16(x):
    hi = x.astype(jnp.bfloat16)
    lo = (x - hi.astype(jnp.float32)).astype(jnp.bfloat16)
    return hi, lo


def _mm3(x, w):
    xh, xl = _split_bf16(x)
    wh, wl = _split_bf16(w)
    a = jnp.concatenate([xh, xl, xh], axis=1)
    b = jnp.concatenate([wh, wh, wl], axis=0)
    return jnp.dot(a, b, preferred_element_type=jnp.float32)


def _pdot(p_bf, x):
    xh, xl = _split_bf16(x)
    return (jnp.dot(p_bf, xh, preferred_element_type=jnp.float32)
            + jnp.dot(p_bf, xl, preferred_element_type=jnp.float32))


_BCAST_DIMS = (((0,), (0,)), ((), ()))


def _pbcast(p_bf, x):
    xh, xl = _split_bf16(x)
    return (lax.dot_general(p_bf, xh, _BCAST_DIMS,
                            preferred_element_type=jnp.float32)
            + lax.dot_general(p_bf, xl, _BCAST_DIMS,
                              preferred_element_type=jnp.float32))


def _tc_embed_body(x_ref, we_ref, be_ref, w0_ref, dg_ref, out_ref):
    wc = _mm3(we_ref[...], w0_ref[...])
    bc = _mm3(be_ref[...], w0_ref[...])
    g = _mm3(x_ref[...], wc) + bc
    _store_halves(out_ref, g * _dinv_from(dg_ref))


_tc_embed = pl.pallas_call(
    _tc_embed_body,
    out_shape=jax.ShapeDtypeStruct((GROWS, HH), jnp.float32),
    compiler_params=_TC_PARAMS,
)


def _norm_pool_half(agg_ref, gt_ref, dg_ref, batch_ref, b_ref, gw_ref,
                    gb_ref, gs_ref):
    dinv = _dinv_from(dg_ref)
    t = dinv * (agg_ref[...] + gt_ref[...]) + b_ref[...]
    p = (lax.broadcasted_iota(jnp.int32, (G, N), 0)
         == batch_ref[...]).astype(jnp.bfloat16)
    cnt = jnp.maximum(jnp.sum(p.astype(jnp.float32), axis=1,
                              keepdims=True), 1.0)
    mean = _pdot(p, t) / cnt
    meanfull = _pbcast(p, mean)
    o = t - meanfull * gs_ref[...]
    var = _pdot(p, o * o) / cnt
    inv_std = lax.rsqrt(var + 1e-5)
    isf = _pbcast(p, inv_std)
    h = gw_ref[...] * o * isf + gb_ref[...]
    h = jnp.where(h >= 0, h, 0.01 * h)
    feats = _pdot(p, h) / cnt
    return h, feats


def _tc_norm_mid_body(agg_ref, gt_ref, dg_ref, batch_ref, b_ref, gw_ref,
                      gb_ref, gs_ref, h_ref, feats_ref):
    h, feats = _norm_pool_half(agg_ref, gt_ref, dg_ref, batch_ref, b_ref,
                               gw_ref, gb_ref, gs_ref)
    h_ref[...] = h
    feats_ref[...] = feats


def _tc_norm_last_body(agg_ref, gt_ref, dg_ref, batch_ref, b_ref, gw_ref,
                       gb_ref, gs_ref, f0_ref, f1_ref, merge_ref):
    _, feats = _norm_pool_half(agg_ref, gt_ref, dg_ref, batch_ref, b_ref,
                               gw_ref, gb_ref, gs_ref)
    merge_ref[...] = (f0_ref[...] + f1_ref[...] + feats) * (1.0 / 3.0)


_half_rows = pl.BlockSpec((N, HH), lambda f: (f, 0))
_half_cols_row = pl.BlockSpec((1, HH), lambda f: (0, f))
_half_cols_g = pl.BlockSpec((G, HH), lambda f: (0, f))
_full_deg = pl.BlockSpec((2 * N, 8), lambda f: (0, 0))
_full_batch = pl.BlockSpec((1, N), lambda f: (0, 0))

_tc_norm_mid = pl.pallas_call(
    _tc_norm_mid_body,
    grid=(2,),
    in_specs=[_half_rows, _half_rows, _full_deg, _full_batch,
              _half_cols_row, _half_cols_row, _half_cols_row, _half_cols_row],
    out_specs=[_half_rows, _half_cols_g],
    out_shape=[
        jax.ShapeDtypeStruct((2 * N, HH), jnp.float32),
        jax.ShapeDtypeStruct((G, H), jnp.float32),
    ],
    compiler_params=_TC_PARAMS,
)

_tc_norm_last = pl.pallas_call(
    _tc_norm_last_body,
    grid=(2,),
    in_specs=[_half_rows, _half_rows, _full_deg, _full_batch,
              _half_cols_row, _half_cols_row, _half_cols_row, _half_cols_row,
              _half_cols_g, _half_cols_g],
    out_specs=_half_cols_g,
    out_shape=jax.ShapeDtypeStruct((G, H), jnp.float32),
    compiler_params=_TC_PARAMS,
)


def _tc_matmul_body(h_ref, w_ref, dg_ref, out_ref):
    gn = (_mm3(h_ref[0:N, :], w_ref[0:HH, :])
          + _mm3(h_ref[N:2 * N, :], w_ref[HH:H, :]))
    _store_halves(out_ref, gn * _dinv_from(dg_ref))


_tc_matmul = pl.pallas_call(
    _tc_matmul_body,
    out_shape=jax.ShapeDtypeStruct((GROWS, HH), jnp.float32),
    compiler_params=_TC_PARAMS,
)


def kernel(x, edge_index, batch, W_emb, b_emb, W0, b0, gn_w0, gn_b0, gn_s0,
           W1, b1, gn_w1, gn_b1, gn_s1, W2, b2, gn_w2, gn_b2, gn_s2):
    src = edge_index[0]
    dst = edge_index[1]
    src_t = src.reshape(NS, EPT)
    pad = ((0, 0), (0, EPTP - EPT))
    src0 = jnp.pad(src_t, pad, constant_values=2 * N)
    src1 = jnp.pad(src_t + N, pad, constant_values=2 * N)
    srcp = jnp.concatenate([src0, src1], axis=0).reshape(2 * NS, CT, K)
    dstp = jnp.pad(dst.reshape(NS, EPT), pad,
                   constant_values=DUMMY).reshape(NS, CT, K)

    ones_rows = jnp.ones((K, HH), jnp.float32)
    z128 = jnp.zeros((ZR, HH), jnp.float32)
    batch_r = batch.reshape(1, N)
    be = b_emb.reshape(1, H)
    layer = [
        (W0, b0.reshape(1, H), gn_w0.reshape(1, H), gn_b0.reshape(1, H),
         gn_s0.reshape(1, H)),
        (W1, b1.reshape(1, H), gn_w1.reshape(1, H), gn_b1.reshape(1, H),
         gn_s1.reshape(1, H)),
        (W2, b2.reshape(1, H), gn_w2.reshape(1, H), gn_b2.reshape(1, H),
         gn_s2.reshape(1, H)),
    ]

    degacc = _sc_deg(dstp, ones_rows, z128)[:, 0:8]

    gt0 = _tc_embed(x, W_emb, be, W0, degacc)
    agg0 = _sc_scatter(gt0, srcp, dstp, z128)
    h1, f0 = _tc_norm_mid(agg0, gt0, degacc, batch_r, layer[0][1],
                          layer[0][2], layer[0][3], layer[0][4])
    gt1 = _tc_matmul(h1, W1, degacc)
    agg1 = _sc_scatter(gt1, srcp, dstp, z128)
    h2, f1 = _tc_norm_mid(agg1, gt1, degacc, batch_r, layer[1][1],
                          layer[1][2], layer[1][3], layer[1][4])
    gt2 = _tc_matmul(h2, W2, degacc)
    agg2 = _sc_scatter(gt2, srcp, dstp, z128)
    merge = _tc_norm_last(agg2, gt2, degacc, batch_r, layer[2][1],
                          layer[2][2], layer[2][3], layer[2][4], f0, f1)
    return merge
